# baseline (speedup 1.0000x reference)
import jax
from jax.experimental import pallas as pl


_ROWS_PER_STEP = 1152


def _copy_body(x_ref, o_ref):
    o_ref[...] = x_ref[...]


def kernel(x):
    flat = x.reshape(-1, 128)
    n = flat.shape[0]
    spec = pl.BlockSpec((_ROWS_PER_STEP, 128), lambda i: (i, 0))
    out = pl.pallas_call(
        _copy_body,
        out_shape=jax.ShapeDtypeStruct(flat.shape, flat.dtype),
        grid=(n // _ROWS_PER_STEP,),
        in_specs=[spec],
        out_specs=spec,
    )(flat)
    return out.reshape(x.shape)
